# baseline (speedup 1.0000x reference)
import functools

import jax
import jax.numpy as jnp
from jax import lax
from jax.experimental import pallas as pl
from jax.experimental.pallas import tpu as pltpu
from jax.experimental.pallas import tpu_sc as plsc

N = 10000
E = 320000
D = 128
DE = 16
DO = 64
NC = 2
NS = 16
NW = NC * NS
EPT = E // NW
CH = EPT // 128 + 1
CHP = 80
EPTP = CHP * 128
NPAD = 10240
NB = 1000
EB = 6400
SEG = NPAD // NS
NSTG = 5
CHS = CHP // NSTG
EPS = 1e-16
SLOPE = 0.2

_mesh = plsc.VectorSubcoreMesh(
    core_axis_name="c", subcore_axis_name="s", num_cores=NC, num_subcores=NS)
_sc_params = pltpu.CompilerParams(needs_layout_passes=False)


def _tc_in_body(x_ref, w0_ref, att0_ref, h_ref, st_ref):
  h = jnp.dot(x_ref[...], w0_ref[...], preferred_element_type=jnp.float32)
  h_ref[...] = h
  st_ref[...] = lax.dot_general(h, att0_ref[...][:2],
                                (((1,), (1,)), ((), ())),
                                preferred_element_type=jnp.float32)


def _tc_le_body(ea_ref, we0_ref, att0_ref, we1_ref, att1_ref, le_ref):
  ae0 = lax.dot_general(att0_ref[...][2:3], we0_ref[...],
                        (((1,), (1,)), ((), ())),
                        preferred_element_type=jnp.float32)
  ae1 = lax.dot_general(att1_ref[...][2:3], we1_ref[...],
                        (((1,), (1,)), ((), ())),
                        preferred_element_type=jnp.float32)
  ae = jnp.concatenate([ae0, ae1], axis=0)
  le_ref[...] = lax.dot_general(ae, ea_ref[...],
                                (((1,), (1,)), ((), ())),
                                preferred_element_type=jnp.float32)


def _tc_mid_body(a0_ref, a1_ref, d0_ref, d1_ref, b0_ref, w1_ref, att1_ref,
                 h_ref, st_ref):
  r = 1.0 / (d0_ref[...] + d1_ref[...] + EPS)
  g = (a0_ref[...] + a1_ref[...]) * r + b0_ref[...]
  g = jnp.where(g > 0, g, jnp.exp(g) - 1.0)
  h = jnp.dot(g, w1_ref[...], preferred_element_type=jnp.float32)
  h_ref[...] = h
  st_ref[...] = lax.dot_general(h, att1_ref[...][:2],
                                (((1,), (1,)), ((), ())),
                                preferred_element_type=jnp.float32)


def _tc_out_body(a0_ref, a1_ref, d0_ref, d1_ref, b1_ref, wc_ref, bc_ref,
                 o_ref):
  r = 1.0 / (d0_ref[...] + d1_ref[...] + EPS)
  g = (a0_ref[...] + a1_ref[...]) * r + b1_ref[...]
  g = jnp.where(g > 0, g, jnp.exp(g) - 1.0)
  o_ref[...] = jnp.dot(g, wc_ref[...],
                       preferred_element_type=jnp.float32) + bc_ref[...]


def _sc_logits_body(s_hbm, t_hbm, lep_hbm, srcp_hbm, dstp_hbm,
                    ex_hbm, dp_hbm,
                    s_v, t_v, src_v, dst_v, le_v, ex_v, z_v, denom_sp):
  c = lax.axis_index("c")
  sid = lax.axis_index("s")
  wid = sid * NC + c
  off = sid * SEG

  pltpu.sync_copy(s_hbm, s_v)
  pltpu.sync_copy(t_hbm, t_v)
  pltpu.sync_copy(srcp_hbm.at[wid], src_v)
  pltpu.sync_copy(dstp_hbm.at[wid], dst_v)
  pltpu.sync_copy(lep_hbm.at[wid], le_v)

  def _zb(k, _):
    z_v[pl.ds(k * 16, 16)] = jnp.zeros((16,), jnp.float32)
    return 0
  lax.fori_loop(0, SEG // 16, _zb, 0)
  pltpu.sync_copy(z_v, denom_sp.at[pl.ds(off, SEG)])
  plsc.subcore_barrier()

  def _chunk(j, _):
    for k in range(8):
      sl = pl.ds(k * 16, 16)
      si = src_v[j, sl]
      di = dst_v[j, sl]
      sv = plsc.load_gather(s_v, [si])
      tv = plsc.load_gather(t_v, [di])
      l = sv + tv + le_v[j, sl]
      l = jnp.where(l >= 0, l, l * SLOPE)
      ex_v[j, sl] = jnp.exp(l)
    return 0
  lax.fori_loop(0, CHP, _chunk, 0)

  pltpu.sync_copy(ex_v, ex_hbm.at[wid])

  def _scat(j, _):
    pltpu.sync_copy(ex_v.at[j], denom_sp.at[dst_v.at[j]], add=True)
    return 0
  lax.fori_loop(0, CHP, _scat, 0)
  plsc.subcore_barrier()

  pltpu.sync_copy(denom_sp.at[pl.ds(off, SEG)], z_v)
  pltpu.sync_copy(z_v, dp_hbm.at[c, pl.ds(off, SEG)])


def _sc_agg_body(h_hbm, srcp_hbm, dstp_hbm, exp_hbm,
                 aggp_hbm,
                 src_v, dst_v, ex_v, rows0, rows1,
                 semg0, semg1, sems0, sems1, out_sp):
  c = lax.axis_index("c")
  sid = lax.axis_index("s")
  wid = sid * NC + c
  off = sid * SEG

  def _zr(e, _):
    for q in range(8):
      rows0[e, pl.ds(q * 16, 16)] = jnp.zeros((16,), jnp.float32)
    return 0
  lax.fori_loop(0, 128, _zr, 0)
  for q in range(SEG // 128):
    pltpu.sync_copy(rows0, out_sp.at[pl.ds(off + q * 128, 128)])
  plsc.subcore_barrier()

  def _gather(j, buf, sem):
    pltpu.async_copy(h_hbm.at[src_v.at[j]], buf, sem)

  def _gwait(buf, sem):
    pltpu.make_async_copy(h_hbm.at[src_v.at[0]], buf, sem).wait()

  def _scatter(j, buf, sem):
    pltpu.async_copy(buf, out_sp.at[dst_v.at[j]], sem, add=True)

  def _swait(buf, sem):
    pltpu.make_async_copy(buf, out_sp.at[dst_v.at[0]], sem).wait()

  def _scale(j, buf):
    base = j * 128
    def _row8(e8, _):
      for u in range(8):
        e = e8 * 8 + u
        av = jnp.zeros((16,), jnp.int32) + (base + e)
        a16 = plsc.load_gather(ex_v, [av])
        for q in range(8):
          sl = pl.ds(q * 16, 16)
          buf[e, sl] = buf[e, sl] * a16
      return 0
    lax.fori_loop(0, 16, _row8, 0)

  for h in range(NSTG):
    gsl = pl.ds(h * CHS, CHS)
    pltpu.sync_copy(srcp_hbm.at[wid, gsl], src_v)
    pltpu.sync_copy(dstp_hbm.at[wid, gsl], dst_v)
    pltpu.sync_copy(exp_hbm.at[wid, pl.ds(h * CHS * 128, CHS * 128)], ex_v)
    _gather(0, rows0, semg0)
    _gather(1, rows1, semg1)

    def _pair(it, _):
      j0 = 2 * it
      j1 = j0 + 1
      _gwait(rows0, semg0)
      _scale(j0, rows0)
      _scatter(j0, rows0, sems0)
      @pl.when(j0 + 2 < CHS)
      def _():
        _swait(rows0, sems0)
        _gather(j0 + 2, rows0, semg0)
      _gwait(rows1, semg1)
      _scale(j1, rows1)
      _scatter(j1, rows1, sems1)
      @pl.when(j1 + 2 < CHS)
      def _():
        _swait(rows1, sems1)
        _gather(j1 + 2, rows1, semg1)
      return 0
    lax.fori_loop(0, CHS // 2, _pair, 0)
    _swait(rows0, sems0)
    _swait(rows1, sems1)
  plsc.subcore_barrier()

  for q in range(SEG // 128):
    sl = pl.ds(off + q * 128, 128)
    pltpu.sync_copy(out_sp.at[sl], rows0)
    pltpu.sync_copy(rows0, aggp_hbm.at[c, sl])


_tc_in = pl.pallas_call(
    _tc_in_body,
    grid=(N // NB,),
    in_specs=[
        pl.BlockSpec((NB, D), lambda i: (i, 0)),
        pl.BlockSpec((D, D), lambda i: (0, 0)),
        pl.BlockSpec((3, D), lambda i: (0, 0)),
    ],
    out_specs=[
        pl.BlockSpec((NB, D), lambda i: (i, 0)),
        pl.BlockSpec((NB, 2), lambda i: (i, 0)),
    ],
    out_shape=[
        jax.ShapeDtypeStruct((N, D), jnp.float32),
        jax.ShapeDtypeStruct((N, 2), jnp.float32),
    ],
)

_tc_le = pl.pallas_call(
    _tc_le_body,
    grid=(E // EB,),
    in_specs=[
        pl.BlockSpec((EB, DE), lambda i: (i, 0)),
        pl.BlockSpec((DE, D), lambda i: (0, 0)),
        pl.BlockSpec((3, D), lambda i: (0, 0)),
        pl.BlockSpec((DE, D), lambda i: (0, 0)),
        pl.BlockSpec((3, D), lambda i: (0, 0)),
    ],
    out_specs=pl.BlockSpec((2, EB), lambda i: (0, i)),
    out_shape=jax.ShapeDtypeStruct((2, E), jnp.float32),
)

_tc_mid = pl.pallas_call(
    _tc_mid_body,
    grid=(N // NB,),
    in_specs=[
        pl.BlockSpec((NB, D), lambda i: (i, 0)),
        pl.BlockSpec((NB, D), lambda i: (i, 0)),
        pl.BlockSpec((NB, 1), lambda i: (i, 0)),
        pl.BlockSpec((NB, 1), lambda i: (i, 0)),
        pl.BlockSpec((1, D), lambda i: (0, 0)),
        pl.BlockSpec((D, D), lambda i: (0, 0)),
        pl.BlockSpec((3, D), lambda i: (0, 0)),
    ],
    out_specs=[
        pl.BlockSpec((NB, D), lambda i: (i, 0)),
        pl.BlockSpec((NB, 2), lambda i: (i, 0)),
    ],
    out_shape=[
        jax.ShapeDtypeStruct((N, D), jnp.float32),
        jax.ShapeDtypeStruct((N, 2), jnp.float32),
    ],
)

_tc_out = pl.pallas_call(
    _tc_out_body,
    grid=(N // NB,),
    in_specs=[
        pl.BlockSpec((NB, D), lambda i: (i, 0)),
        pl.BlockSpec((NB, D), lambda i: (i, 0)),
        pl.BlockSpec((NB, 1), lambda i: (i, 0)),
        pl.BlockSpec((NB, 1), lambda i: (i, 0)),
        pl.BlockSpec((1, D), lambda i: (0, 0)),
        pl.BlockSpec((D, DO), lambda i: (0, 0)),
        pl.BlockSpec((1, DO), lambda i: (0, 0)),
    ],
    out_specs=pl.BlockSpec((NB, DO), lambda i: (i, 0)),
    out_shape=jax.ShapeDtypeStruct((N, DO), jnp.float32),
)

_sc_logits = pl.kernel(
    _sc_logits_body,
    out_type=(
        jax.ShapeDtypeStruct((NW, CHP, 128), jnp.float32),
        jax.ShapeDtypeStruct((NC, NPAD), jnp.float32),
    ),
    mesh=_mesh,
    scratch_types=[
        pltpu.VMEM((N,), jnp.float32),
        pltpu.VMEM((N,), jnp.float32),
        pltpu.VMEM((CHP, 128), jnp.int32),
        pltpu.VMEM((CHP, 128), jnp.int32),
        pltpu.VMEM((CHP, 128), jnp.float32),
        pltpu.VMEM((CHP, 128), jnp.float32),
        pltpu.VMEM((SEG,), jnp.float32),
        pltpu.VMEM_SHARED((NPAD,), jnp.float32),
    ],
    compiler_params=_sc_params,
)

_sc_agg = pl.kernel(
    _sc_agg_body,
    out_type=jax.ShapeDtypeStruct((NC, NPAD, D), jnp.float32),
    mesh=_mesh,
    scratch_types=[
        pltpu.VMEM((CHS, 128), jnp.int32),
        pltpu.VMEM((CHS, 128), jnp.int32),
        pltpu.VMEM((CHS * 128,), jnp.float32),
        pltpu.VMEM((128, D), jnp.float32),
        pltpu.VMEM((128, D), jnp.float32),
        pltpu.SemaphoreType.DMA,
        pltpu.SemaphoreType.DMA,
        pltpu.SemaphoreType.DMA,
        pltpu.SemaphoreType.DMA,
        pltpu.VMEM_SHARED((NPAD, D), jnp.float32),
    ],
    compiler_params=_sc_params,
)


def _pad_tiles(a, pad_value):
  a2 = a.reshape(NW, EPT)
  a2 = jnp.pad(a2, ((0, 0), (0, EPTP - EPT)), constant_values=pad_value)
  return a2.reshape(NW, CHP, 128)


def kernel(x, edge_index, edge_attr, W0, We0, att0, b0, W1, We1, att1, b1,
           Wc, bc):
  src = edge_index[0].astype(jnp.int32)
  dst = edge_index[1].astype(jnp.int32)
  srcp = _pad_tiles(src, 0)
  dstp = _pad_tiles(dst, N)

  h0, st0 = _tc_in(x, W0, att0)
  le01 = _tc_le(edge_attr, We0, att0, We1, att1)
  s0 = st0[:, 0]
  t0 = st0[:, 1]
  lep0 = _pad_tiles(le01[0], 0.0)
  lep1 = _pad_tiles(le01[1], 0.0)

  ex0, dp0 = _sc_logits(s0, t0, lep0, srcp, dstp)
  aggp0 = _sc_agg(h0, srcp, dstp, ex0.reshape(NW, EPTP))

  h1, st1 = _tc_mid(aggp0[0], aggp0[1], dp0[0].reshape(NPAD, 1),
                    dp0[1].reshape(NPAD, 1), b0.reshape(1, D), W1, att1)
  ex1, dp1 = _sc_logits(st1[:, 0], st1[:, 1], lep1, srcp, dstp)
  aggp1 = _sc_agg(h1, srcp, dstp, ex1.reshape(NW, EPTP))

  out = _tc_out(aggp1[0], aggp1[1], dp1[0].reshape(NPAD, 1),
                dp1[1].reshape(NPAD, 1), b1.reshape(1, D), Wc,
                bc.reshape(1, DO))
  return out


# speedup vs baseline: 1.0085x; 1.0085x over previous
import functools

import jax
import jax.numpy as jnp
import numpy as np
from jax import lax
from jax.experimental import pallas as pl
from jax.experimental.pallas import tpu as pltpu
from jax.experimental.pallas import tpu_sc as plsc

N = 10000
E = 320000
D = 128
DE = 16
DO = 64
NC = 2
NS = 16
NW = NC * NS
EPT = E // NW
CH = EPT // 128 + 1
CHP = 80
EPTP = CHP * 128
NPAD = 10240
NB = 1000
EB = 6400
SEG = NPAD // NS
NSTG = 5
CHS = CHP // NSTG
EPS = 1e-16
SLOPE = 0.2

_mesh = plsc.VectorSubcoreMesh(
    core_axis_name="c", subcore_axis_name="s", num_cores=NC, num_subcores=NS)
_sc_params = pltpu.CompilerParams(needs_layout_passes=False)
_sc_params_flat = pltpu.CompilerParams(needs_layout_passes=False,
                                       use_tc_tiling_on_sc=False)


def _tc_in_body(x_ref, w0_ref, att0_ref, h_ref, st_ref):
  h = jnp.dot(x_ref[...], w0_ref[...], preferred_element_type=jnp.float32)
  h_ref[...] = h
  st_ref[...] = lax.dot_general(h, att0_ref[...][:2],
                                (((1,), (1,)), ((), ())),
                                preferred_element_type=jnp.float32)


def _tc_le_body(ea_ref, we0_ref, att0_ref, we1_ref, att1_ref, le_ref):
  ae0 = lax.dot_general(att0_ref[...][2:3], we0_ref[...],
                        (((1,), (1,)), ((), ())),
                        preferred_element_type=jnp.float32)
  ae1 = lax.dot_general(att1_ref[...][2:3], we1_ref[...],
                        (((1,), (1,)), ((), ())),
                        preferred_element_type=jnp.float32)
  ae = jnp.concatenate([ae0, ae1], axis=0)
  le_ref[...] = lax.dot_general(ae, ea_ref[...],
                                (((1,), (1,)), ((), ())),
                                preferred_element_type=jnp.float32)


def _tc_mid_body(a0_ref, a1_ref, d0_ref, d1_ref, b0_ref, w1_ref, att1_ref,
                 h_ref, st_ref):
  r = 1.0 / (d0_ref[...] + d1_ref[...] + EPS)
  g = (a0_ref[...] + a1_ref[...]) * r + b0_ref[...]
  g = jnp.where(g > 0, g, jnp.exp(g) - 1.0)
  h = jnp.dot(g, w1_ref[...], preferred_element_type=jnp.float32)
  h_ref[...] = h
  st_ref[...] = lax.dot_general(h, att1_ref[...][:2],
                                (((1,), (1,)), ((), ())),
                                preferred_element_type=jnp.float32)


def _tc_out_body(a0_ref, a1_ref, d0_ref, d1_ref, b1_ref, wc_ref, bc_ref,
                 o_ref):
  r = 1.0 / (d0_ref[...] + d1_ref[...] + EPS)
  g = (a0_ref[...] + a1_ref[...]) * r + b1_ref[...]
  g = jnp.where(g > 0, g, jnp.exp(g) - 1.0)
  o_ref[...] = jnp.dot(g, wc_ref[...],
                       preferred_element_type=jnp.float32) + bc_ref[...]


def _sc_logits_body(s_hbm, t_hbm, lep_hbm, srcp_hbm, dstp_hbm,
                    ex_hbm, dp_hbm,
                    s_v, t_v, src_v, dst_v, le_v, ex_v, z_v, denom_sp):
  c = lax.axis_index("c")
  sid = lax.axis_index("s")
  wid = sid * NC + c
  off = sid * SEG

  pltpu.sync_copy(s_hbm, s_v)
  pltpu.sync_copy(t_hbm, t_v)
  pltpu.sync_copy(srcp_hbm.at[wid], src_v)
  pltpu.sync_copy(dstp_hbm.at[wid], dst_v)
  pltpu.sync_copy(lep_hbm.at[wid], le_v)

  def _zb(k, _):
    z_v[pl.ds(k * 16, 16)] = jnp.zeros((16,), jnp.float32)
    return 0
  lax.fori_loop(0, SEG // 16, _zb, 0)
  pltpu.sync_copy(z_v, denom_sp.at[pl.ds(off, SEG)])
  plsc.subcore_barrier()

  def _chunk(j, _):
    for k in range(8):
      sl = pl.ds(k * 16, 16)
      si = src_v[j, sl]
      di = dst_v[j, sl]
      sv = plsc.load_gather(s_v, [si])
      tv = plsc.load_gather(t_v, [di])
      l = sv + tv + le_v[j, sl]
      l = jnp.where(l >= 0, l, l * SLOPE)
      ex_v[j, sl] = jnp.exp(l)
    return 0
  lax.fori_loop(0, CHP, _chunk, 0)

  pltpu.sync_copy(ex_v, ex_hbm.at[wid])

  def _scat(j, _):
    pltpu.sync_copy(ex_v.at[j], denom_sp.at[dst_v.at[j]], add=True)
    return 0
  lax.fori_loop(0, CHP, _scat, 0)
  plsc.subcore_barrier()

  pltpu.sync_copy(denom_sp.at[pl.ds(off, SEG)], z_v)
  pltpu.sync_copy(z_v, dp_hbm.at[c, pl.ds(off, SEG)])


def _sc_agg_body(hb_hbm, srcp_hbm, dstp_hbm, exp_hbm,
                 aggp_hbm,
                 src_v, dst_v, ex_v, grows, frows,
                 semg0, semg1, sems, out_sp):
  c = lax.axis_index("c")
  sid = lax.axis_index("s")
  wid = sid * NC + c
  off = sid * SEG

  def _zr(e, _):
    for q in range(8):
      frows[e, pl.ds(q * 16, 16)] = jnp.zeros((16,), jnp.float32)
    return 0
  lax.fori_loop(0, 128, _zr, 0)
  for q in range(SEG // 128):
    pltpu.sync_copy(frows, out_sp.at[pl.ds(off + q * 128, 128)])
  plsc.subcore_barrier()

  def _gather(j, slot, sem):
    pltpu.async_copy(hb_hbm.at[src_v.at[j]], grows.at[pl.ds(slot * 128, 128)],
                     sem)

  def _gwait(slot, sem):
    pltpu.make_async_copy(hb_hbm.at[src_v.at[0]],
                          grows.at[pl.ds(slot * 128, 128)], sem).wait()

  def _scatter(j):
    pltpu.async_copy(frows, out_sp.at[dst_v.at[j]], sems, add=True)

  def _swait():
    pltpu.make_async_copy(frows, out_sp.at[dst_v.at[0]], sems).wait()

  mask_hi = jnp.full((16,), -65536, jnp.int32)
  def _unpack_scale(j, slot):
    base = j * 128
    srow = slot * 128
    def _row4(e4, _):
      for u in range(4):
        e = e4 * 4 + u
        av = jnp.zeros((16,), jnp.int32) + (base + e)
        a16 = plsc.load_gather(ex_v, [av])
        for q in range(4):
          v = grows[srow + e, pl.ds(q * 16, 16)]
          ev = plsc.bitcast(v << 16, jnp.float32)
          ov = plsc.bitcast(v & mask_hi, jnp.float32)
          frows[e, pl.ds(q * 32, 16)] = ev * a16
          frows[e, pl.ds(q * 32 + 16, 16)] = ov * a16
      return 0
    lax.fori_loop(0, 32, _row4, 0)

  for h in range(NSTG):
    gsl = pl.ds(h * CHS, CHS)
    pltpu.sync_copy(srcp_hbm.at[wid, gsl], src_v)
    pltpu.sync_copy(dstp_hbm.at[wid, gsl], dst_v)
    pltpu.sync_copy(exp_hbm.at[wid, pl.ds(h * CHS * 128, CHS * 128)], ex_v)
    _gather(0, 0, semg0)
    _gather(1, 1, semg1)

    def _pair(it, _):
      j0 = 2 * it
      j1 = j0 + 1
      _gwait(0, semg0)
      @pl.when(j0 > 0)
      def _():
        _swait()
      _unpack_scale(j0, 0)
      _scatter(j0)
      @pl.when(j0 + 2 < CHS)
      def _():
        _gather(j0 + 2, 0, semg0)
      _gwait(1, semg1)
      _swait()
      _unpack_scale(j1, 1)
      _scatter(j1)
      @pl.when(j1 + 2 < CHS)
      def _():
        _gather(j1 + 2, 1, semg1)
      return 0
    lax.fori_loop(0, CHS // 2, _pair, 0)
    _swait()
  plsc.subcore_barrier()

  for q in range(SEG // 128):
    sl = pl.ds(off + q * 128, 128)
    pltpu.sync_copy(out_sp.at[sl], frows)
    pltpu.sync_copy(frows, aggp_hbm.at[c, sl])


_tc_in = pl.pallas_call(
    _tc_in_body,
    grid=(N // NB,),
    in_specs=[
        pl.BlockSpec((NB, D), lambda i: (i, 0)),
        pl.BlockSpec((D, D), lambda i: (0, 0)),
        pl.BlockSpec((3, D), lambda i: (0, 0)),
    ],
    out_specs=[
        pl.BlockSpec((NB, D), lambda i: (i, 0)),
        pl.BlockSpec((NB, 2), lambda i: (i, 0)),
    ],
    out_shape=[
        jax.ShapeDtypeStruct((N, D), jnp.float32),
        jax.ShapeDtypeStruct((N, 2), jnp.float32),
    ],
)

_tc_le = pl.pallas_call(
    _tc_le_body,
    grid=(E // EB,),
    in_specs=[
        pl.BlockSpec((EB, DE), lambda i: (i, 0)),
        pl.BlockSpec((DE, D), lambda i: (0, 0)),
        pl.BlockSpec((3, D), lambda i: (0, 0)),
        pl.BlockSpec((DE, D), lambda i: (0, 0)),
        pl.BlockSpec((3, D), lambda i: (0, 0)),
    ],
    out_specs=pl.BlockSpec((2, EB), lambda i: (0, i)),
    out_shape=jax.ShapeDtypeStruct((2, E), jnp.float32),
)

_tc_mid = pl.pallas_call(
    _tc_mid_body,
    grid=(N // NB,),
    in_specs=[
        pl.BlockSpec((NB, D), lambda i: (i, 0)),
        pl.BlockSpec((NB, D), lambda i: (i, 0)),
        pl.BlockSpec((NB, 1), lambda i: (i, 0)),
        pl.BlockSpec((NB, 1), lambda i: (i, 0)),
        pl.BlockSpec((1, D), lambda i: (0, 0)),
        pl.BlockSpec((D, D), lambda i: (0, 0)),
        pl.BlockSpec((3, D), lambda i: (0, 0)),
    ],
    out_specs=[
        pl.BlockSpec((NB, D), lambda i: (i, 0)),
        pl.BlockSpec((NB, 2), lambda i: (i, 0)),
    ],
    out_shape=[
        jax.ShapeDtypeStruct((N, D), jnp.float32),
        jax.ShapeDtypeStruct((N, 2), jnp.float32),
    ],
)

_tc_out = pl.pallas_call(
    _tc_out_body,
    grid=(N // NB,),
    in_specs=[
        pl.BlockSpec((NB, D), lambda i: (i, 0)),
        pl.BlockSpec((NB, D), lambda i: (i, 0)),
        pl.BlockSpec((NB, 1), lambda i: (i, 0)),
        pl.BlockSpec((NB, 1), lambda i: (i, 0)),
        pl.BlockSpec((1, D), lambda i: (0, 0)),
        pl.BlockSpec((D, DO), lambda i: (0, 0)),
        pl.BlockSpec((1, DO), lambda i: (0, 0)),
    ],
    out_specs=pl.BlockSpec((NB, DO), lambda i: (i, 0)),
    out_shape=jax.ShapeDtypeStruct((N, DO), jnp.float32),
)

_sc_logits = pl.kernel(
    _sc_logits_body,
    out_type=(
        jax.ShapeDtypeStruct((NW, CHP, 128), jnp.float32),
        jax.ShapeDtypeStruct((NC, NPAD), jnp.float32),
    ),
    mesh=_mesh,
    scratch_types=[
        pltpu.VMEM((N,), jnp.float32),
        pltpu.VMEM((N,), jnp.float32),
        pltpu.VMEM((CHP, 128), jnp.int32),
        pltpu.VMEM((CHP, 128), jnp.int32),
        pltpu.VMEM((CHP, 128), jnp.float32),
        pltpu.VMEM((CHP, 128), jnp.float32),
        pltpu.VMEM((SEG,), jnp.float32),
        pltpu.VMEM_SHARED((NPAD,), jnp.float32),
    ],
    compiler_params=_sc_params,
)

_sc_agg = pl.kernel(
    _sc_agg_body,
    out_type=jax.ShapeDtypeStruct((NC, NPAD, D), jnp.float32),
    mesh=_mesh,
    scratch_types=[
        pltpu.VMEM((CHS, 128), jnp.int32),
        pltpu.VMEM((CHS, 128), jnp.int32),
        pltpu.VMEM((CHS * 128,), jnp.float32),
        pltpu.VMEM((256, D // 2), jnp.int32),
        pltpu.VMEM((128, D), jnp.float32),
        pltpu.SemaphoreType.DMA,
        pltpu.SemaphoreType.DMA,
        pltpu.SemaphoreType.DMA,
        pltpu.VMEM_SHARED((NPAD, D), jnp.float32),
    ],
    compiler_params=_sc_params_flat,
)


def _pad_tiles(a, pad_value):
  a2 = a.reshape(NW, EPT)
  a2 = jnp.pad(a2, ((0, 0), (0, EPTP - EPT)), constant_values=pad_value)
  return a2.reshape(NW, CHP, 128)


_PERM = np.array(
    [32 * q + 2 * l + w
     for q in range(4) for w in range(2) for l in range(16)], np.int32)


def kernel(x, edge_index, edge_attr, W0, We0, att0, b0, W1, We1, att1, b1,
           Wc, bc):
  src = edge_index[0].astype(jnp.int32)
  dst = edge_index[1].astype(jnp.int32)
  srcp = _pad_tiles(src, 0)
  dstp = _pad_tiles(dst, N)

  h0, st0 = _tc_in(x, W0, att0)
  le01 = _tc_le(edge_attr, We0, att0, We1, att1)
  s0 = st0[:, 0]
  t0 = st0[:, 1]
  lep0 = _pad_tiles(le01[0], 0.0)
  lep1 = _pad_tiles(le01[1], 0.0)

  def _packh(h):
    return jax.lax.bitcast_convert_type(
        h.astype(jnp.bfloat16).reshape(N, D // 2, 2), jnp.int32)

  ex0, dp0 = _sc_logits(s0, t0, lep0, srcp, dstp)
  aggp0 = _sc_agg(_packh(h0), srcp, dstp, ex0.reshape(NW, EPTP))

  h1, st1 = _tc_mid(aggp0[0], aggp0[1], dp0[0].reshape(NPAD, 1),
                    dp0[1].reshape(NPAD, 1), b0[_PERM].reshape(1, D),
                    W1[_PERM], att1)
  ex1, dp1 = _sc_logits(st1[:, 0], st1[:, 1], lep1, srcp, dstp)
  aggp1 = _sc_agg(_packh(h1), srcp, dstp, ex1.reshape(NW, EPTP))

  out = _tc_out(aggp1[0], aggp1[1], dp1[0].reshape(NPAD, 1),
                dp1[1].reshape(NPAD, 1), b1[_PERM].reshape(1, D), Wc[_PERM],
                bc.reshape(1, DO))
  return out
